# Initial kernel scaffold; baseline (speedup 1.0000x reference)
import jax
import jax.numpy as jnp
from jax.experimental import pallas as pl


def kernel(x, emb_weight):
    raise NotImplementedError("write your pallas kernel here")


# speedup vs baseline: 4.0869x; 4.0869x over previous
import functools

import jax
import jax.numpy as jnp
from jax import lax
from jax.experimental import pallas as pl
from jax.experimental.pallas import tpu as pltpu
from jax.experimental.pallas import tpu_sc as plsc

NC = 2
NS = 16
NW = NC * NS
CHUNK = 128


@functools.partial(jax.jit, static_argnames=("n_chunks", "n_per_w", "d"))
def _emb_lookup(emb_weight, idx2d, n_chunks, n_per_w, d):
    mesh = plsc.VectorSubcoreMesh(core_axis_name="c", subcore_axis_name="s")

    @functools.partial(
        pl.kernel,
        mesh=mesh,
        compiler_params=pltpu.CompilerParams(use_tc_tiling_on_sc=False),
        out_type=jax.ShapeDtypeStruct((n_chunks * CHUNK, d), jnp.float32),
        scratch_types=[
            pltpu.VMEM((n_per_w, CHUNK), jnp.int32),
            pltpu.VMEM((CHUNK, d), jnp.float32),
            pltpu.SemaphoreType.DMA,
        ],
    )
    def k(table_hbm, idx_hbm, out_hbm, idx_v, rows_v, gsem):
        wid = lax.axis_index("s") * NC + lax.axis_index("c")
        base_chunk = wid * n_per_w
        pltpu.sync_copy(idx_hbm.at[wid], idx_v)

        def body(j, carry):
            pltpu.async_copy(table_hbm.at[idx_v.at[j]], rows_v, gsem).wait()
            row0 = pl.multiple_of((base_chunk + j) * CHUNK, CHUNK)
            pltpu.sync_copy(rows_v, out_hbm.at[pl.ds(row0, CHUNK)])
            return carry

        lax.fori_loop(0, n_per_w, body, 0)

    return k(emb_weight, idx2d)


def kernel(x, emb_weight):
    b, s = x.shape
    v, d = emb_weight.shape
    n = b * s
    n_chunks = n // CHUNK
    n_per_w = n_chunks // NW
    idx2d = x.reshape(NW, n_per_w, CHUNK).astype(jnp.int32)
    out = _emb_lookup(emb_weight, idx2d, n_chunks, n_per_w, d)
    return out.reshape(b, s, d)


# speedup vs baseline: 4.6493x; 1.1376x over previous
import functools

import jax
import jax.numpy as jnp
from jax import lax
from jax.experimental import pallas as pl
from jax.experimental.pallas import tpu as pltpu
from jax.experimental.pallas import tpu_sc as plsc

NC = 2
NS = 16
NW = NC * NS
CHUNK = 128
NBUF = 5


@functools.partial(jax.jit, static_argnames=("n_chunks", "n_per_w", "d"))
def _emb_lookup(emb_weight, idx3d, n_chunks, n_per_w, d):
    mesh = plsc.VectorSubcoreMesh(core_axis_name="c", subcore_axis_name="s")
    n_groups = n_per_w // NBUF

    @functools.partial(
        pl.kernel,
        mesh=mesh,
        compiler_params=pltpu.CompilerParams(use_tc_tiling_on_sc=False),
        out_type=jax.ShapeDtypeStruct((n_chunks * CHUNK, d), jnp.float32),
        scratch_types=(
            [pltpu.VMEM((n_per_w, CHUNK), jnp.int32)]
            + [pltpu.VMEM((CHUNK, d), jnp.float32) for _ in range(NBUF)]
            + [pltpu.SemaphoreType.DMA for _ in range(2 * NBUF)]
        ),
    )
    def k(table_hbm, idx_hbm, out_hbm, idx_v, *bufs):
        rows = bufs[:NBUF]
        gsem = bufs[NBUF:2 * NBUF]
        osem = bufs[2 * NBUF:]
        wid = lax.axis_index("s") * NC + lax.axis_index("c")
        base_chunk = wid * n_per_w
        pltpu.sync_copy(idx_hbm.at[wid], idx_v)

        def gather_start(j, b):
            pltpu.async_copy(table_hbm.at[idx_v.at[j]], rows[b], gsem[b])

        def gather_wait(b):
            pltpu.make_async_copy(
                table_hbm.at[idx_v.at[0]], rows[b], gsem[b]).wait()

        def out_start(j, b):
            row0 = pl.multiple_of((base_chunk + j) * CHUNK, CHUNK)
            pltpu.async_copy(rows[b], out_hbm.at[pl.ds(row0, CHUNK)], osem[b])

        def out_wait(b):
            pltpu.make_async_copy(
                rows[b], out_hbm.at[pl.ds(0, CHUNK)], osem[b]).wait()

        for b in range(NBUF):
            gather_start(b, b)

        def outer(t, carry):
            base_j = t * NBUF
            for b in range(NBUF):
                gather_wait(b)
                out_start(base_j + b, b)
            for b in range(NBUF):
                out_wait(b)
                gather_start(base_j + NBUF + b, b)
            return carry

        lax.fori_loop(0, n_groups - 1, outer, 0)

        base_j = (n_groups - 1) * NBUF
        for b in range(NBUF):
            gather_wait(b)
            out_start(base_j + b, b)
        for b in range(NBUF):
            out_wait(b)

    return k(emb_weight, idx3d)


def kernel(x, emb_weight):
    b, s = x.shape
    v, d = emb_weight.shape
    n = b * s
    n_chunks = n // CHUNK
    n_per_w = n_chunks // NW
    idx3d = x.reshape(NW, n_per_w, CHUNK).astype(jnp.int32)
    out = _emb_lookup(emb_weight, idx3d, n_chunks, n_per_w, d)
    return out.reshape(b, s, d)
